# Initial kernel scaffold; baseline (speedup 1.0000x reference)
import jax
import jax.numpy as jnp
from jax.experimental import pallas as pl


def kernel(x, edge_index, Wl1, bl1, Wr1, Wl2, bl2, Wr2, g1, b1, rm1, rv1, g2, b2, rm2, rv2, Wo, bo):
    raise NotImplementedError("write your pallas kernel here")


# speedup vs baseline: 5.7399x; 5.7399x over previous
import functools

import jax
import jax.numpy as jnp
from jax import lax
from jax.experimental import pallas as pl
from jax.experimental.pallas import tpu as pltpu
from jax.experimental.pallas import tpu_sc as plsc

_NC = 2
_NS = 16
_NW = _NC * _NS
_L = 16
_CHUNK = 128
_ZB = 64

_HIGH = lax.Precision.HIGHEST


def _round_up(a, b):
    return -(-a // b) * b


def _make_sc_segsum(n, h, n_pad, e_pad, with_cnt):
    pw = e_pad // _NW
    nchunk = pw // _CHUNK
    rps = n_pad // _NS
    nzb = rps // _ZB
    mesh = plsc.VectorSubcoreMesh(core_axis_name="c", subcore_axis_name="s")

    out_type = [jax.ShapeDtypeStruct((_NC, n_pad, h), jnp.float32)]
    scratch = [
        pltpu.VMEM_SHARED((n_pad, h), jnp.float32),
        pltpu.VMEM((_ZB, h), jnp.float32),
        pltpu.VMEM((_CHUNK,), jnp.int32),
        pltpu.VMEM((_CHUNK,), jnp.int32),
        pltpu.VMEM((_CHUNK, h), jnp.float32),
        pltpu.SemaphoreType.DMA,
    ]
    if with_cnt:
        out_type.append(jax.ShapeDtypeStruct((_NC, n_pad, _L), jnp.float32))
        scratch += [
            pltpu.VMEM_SHARED((n_pad, _L), jnp.float32),
            pltpu.VMEM((_ZB, _L), jnp.float32),
            pltpu.VMEM((_CHUNK, _L), jnp.float32),
        ]

    @functools.partial(
        pl.kernel, mesh=mesh, out_type=tuple(out_type),
        scratch_types=scratch,
        compiler_params=pltpu.CompilerParams(use_tc_tiling_on_sc=False))
    def sc_kernel(y_hbm, src_hbm, dst_hbm, *refs):
        if with_cnt:
            (out_hbm, cnt_hbm, acc_sh, zero_v, src_v, dst_v, rows_v, sem,
             cnt_sh, zcnt_v, ones_v) = refs
        else:
            (out_hbm, acc_sh, zero_v, src_v, dst_v, rows_v, sem) = refs

        c = lax.axis_index("c")
        s = lax.axis_index("s")
        w = c * _NS + s

        zf = jnp.zeros((_L,), jnp.float32)

        @pl.loop(0, _ZB)
        def _(i):
            for j in range(0, h, _L):
                zero_v[i, pl.ds(j, _L)] = zf

        zbase = s * rps

        @pl.loop(0, nzb)
        def _(k):
            pltpu.sync_copy(zero_v, acc_sh.at[pl.ds(zbase + k * _ZB, _ZB)])

        if with_cnt:
            of = jnp.full((_L,), 1.0, jnp.float32)

            @pl.loop(0, _ZB)
            def _(i):
                zcnt_v[i, pl.ds(0, _L)] = zf

            @pl.loop(0, _CHUNK)
            def _(i):
                ones_v[i, pl.ds(0, _L)] = of

            @pl.loop(0, nzb)
            def _(k):
                pltpu.sync_copy(zcnt_v, cnt_sh.at[pl.ds(zbase + k * _ZB, _ZB)])

        plsc.subcore_barrier()

        ebase = w * pw

        @pl.loop(0, nchunk)
        def _(j):
            off = ebase + j * _CHUNK
            pltpu.sync_copy(src_hbm.at[pl.ds(off, _CHUNK)], src_v)
            pltpu.async_copy(y_hbm.at[src_v], rows_v, sem).wait()
            pltpu.sync_copy(dst_hbm.at[pl.ds(off, _CHUNK)], dst_v)
            pltpu.sync_copy(rows_v, acc_sh.at[dst_v], add=True)
            if with_cnt:
                pltpu.sync_copy(ones_v, cnt_sh.at[dst_v], add=True)

        plsc.subcore_barrier()

        obase = s * rps
        pltpu.sync_copy(acc_sh.at[pl.ds(obase, rps)],
                        out_hbm.at[c].at[pl.ds(obase, rps)])
        if with_cnt:
            pltpu.sync_copy(cnt_sh.at[pl.ds(obase, rps)],
                            cnt_hbm.at[c].at[pl.ds(obase, rps)])

    return sc_kernel


def _proj2(x, wa, wb):
    m = x.shape[0]
    ha = wa.shape[1]
    hb = wb.shape[1]

    def body(x_ref, wa_ref, wb_ref, ya_ref, yb_ref):
        xv = x_ref[...]
        ya_ref[...] = lax.dot(xv, wa_ref[...], precision=_HIGH)
        yb_ref[...] = lax.dot(xv, wb_ref[...], precision=_HIGH)

    return pl.pallas_call(
        body,
        out_shape=(jax.ShapeDtypeStruct((m, ha), jnp.float32),
                   jax.ShapeDtypeStruct((m, hb), jnp.float32)),
    )(x, wa, wb)


def _combine(p_ref, c_ref, r_ref, a_ref, cv_ref, n, h):
    agg = p_ref[0, :n, :] + p_ref[1, :n, :]
    ct = c_ref[0, :n, :] + c_ref[1, :n, :]
    inv = 1.0 / jnp.maximum(ct, 1.0)
    inv_h = jnp.concatenate([inv] * (h // _L), axis=1)
    return jnp.maximum((agg * inv_h + r_ref[...]) * a_ref[...] + cv_ref[...],
                       0.0)


def _mid_layer(p, cnt, r, a, cv, wl, wr, n):
    h = r.shape[1]

    def body(p_ref, c_ref, r_ref, a_ref, cv_ref, wl_ref, wr_ref, y_ref, rr_ref):
        hid = _combine(p_ref, c_ref, r_ref, a_ref, cv_ref, n, h)
        y_ref[...] = lax.dot(hid, wl_ref[...], precision=_HIGH)
        rr_ref[...] = lax.dot(hid, wr_ref[...], precision=_HIGH)

    return pl.pallas_call(
        body,
        out_shape=(jax.ShapeDtypeStruct((n, wl.shape[1]), jnp.float32),
                   jax.ShapeDtypeStruct((n, wr.shape[1]), jnp.float32)),
    )(p, cnt, r, a, cv, wl, wr)


def _final_layer(p, cnt, r, a, cv, wo, bo, n):
    h = r.shape[1]

    def body(p_ref, c_ref, r_ref, a_ref, cv_ref, wo_ref, bo_ref, o_ref):
        hid = _combine(p_ref, c_ref, r_ref, a_ref, cv_ref, n, h)
        o_ref[...] = (jnp.sum(hid * wo_ref[...], axis=1, keepdims=True)
                      + bo_ref[...])

    return pl.pallas_call(
        body,
        out_shape=jax.ShapeDtypeStruct((n, 1), jnp.float32),
    )(p, cnt, r, a, cv, wo, bo)


def kernel(x, edge_index, Wl1, bl1, Wr1, Wl2, bl2, Wr2,
           g1, b1, rm1, rv1, g2, b2, rm2, rv2, Wo, bo):
    n, d = x.shape
    e = edge_index.shape[1]
    h = Wl1.shape[0]

    e_pad = _round_up(e, _NW * _CHUNK)
    n_pad = _round_up(n, _NS * _ZB)
    if e_pad > e and n_pad == n:
        n_pad += _NS * _ZB
    src = edge_index[0]
    dst = edge_index[1]
    if e_pad > e:
        pad = e_pad - e
        src = jnp.concatenate([src, jnp.zeros((pad,), jnp.int32)])
        dst = jnp.concatenate([dst, jnp.full((pad,), n, jnp.int32)])

    a1 = g1 / jnp.sqrt(rv1 + 1e-5)
    c1 = b1 + (bl1 - rm1) * a1
    a2 = g2 / jnp.sqrt(rv2 + 1e-5)
    c2 = b2 + (bl2 - rm2) * a2

    sc1 = _make_sc_segsum(n, h, n_pad, e_pad, with_cnt=True)
    sc2 = _make_sc_segsum(n, h, n_pad, e_pad, with_cnt=False)

    y1, r1 = _proj2(x, Wl1.T, Wr1.T)
    p1, cnt = sc1(y1, src, dst)
    y2, r2 = _mid_layer(p1, cnt, r1, a1.reshape(1, h), c1.reshape(1, h),
                        Wl2.T, Wr2.T, n)
    (p2,) = sc2(y2, src, dst)
    out = _final_layer(p2, cnt, r2, a2.reshape(1, h), c2.reshape(1, h),
                       Wo.reshape(1, h), bo.reshape(1, 1), n)
    return out


# speedup vs baseline: 6.0128x; 1.0476x over previous
import functools

import jax
import jax.numpy as jnp
from jax import lax
from jax.experimental import pallas as pl
from jax.experimental.pallas import tpu as pltpu
from jax.experimental.pallas import tpu_sc as plsc

_NC = 2
_NS = 16
_NW = _NC * _NS
_L = 16
_CHUNK = 128
_ZB = 64

_HIGH = lax.Precision.HIGHEST


def _round_up(a, b):
    return -(-a // b) * b


_NBUF = 5


def _make_sc_segsum(n, h, n_pad, e_pad, with_cnt):
    pw = e_pad // _NW
    nchunk = pw // _CHUNK
    ngroup = nchunk // _NBUF
    rps = n_pad // _NS
    nzb = rps // _ZB
    mesh = plsc.VectorSubcoreMesh(core_axis_name="c", subcore_axis_name="s")

    out_type = [jax.ShapeDtypeStruct((_NC, n_pad, h), jnp.float32)]
    scratch = [
        pltpu.VMEM_SHARED((n_pad, h), jnp.float32),
        pltpu.VMEM((_ZB, h), jnp.float32),
        pltpu.VMEM((nchunk, _CHUNK), jnp.int32),
        pltpu.VMEM((nchunk, _CHUNK), jnp.int32),
        pltpu.VMEM((_NBUF * _CHUNK, h), jnp.float32),
        pltpu.SemaphoreType.DMA((_NBUF,)),
        pltpu.SemaphoreType.DMA((_NBUF,)),
    ]
    if with_cnt:
        out_type.append(jax.ShapeDtypeStruct((_NC, n_pad, _L), jnp.float32))
        scratch += [
            pltpu.VMEM_SHARED((n_pad, _L), jnp.float32),
            pltpu.VMEM((_ZB, _L), jnp.float32),
            pltpu.VMEM((_CHUNK, _L), jnp.float32),
            pltpu.SemaphoreType.DMA((_NBUF,)),
        ]

    @functools.partial(
        pl.kernel, mesh=mesh, out_type=tuple(out_type),
        scratch_types=scratch,
        compiler_params=pltpu.CompilerParams(use_tc_tiling_on_sc=False))
    def sc_kernel(y_hbm, src_hbm, dst_hbm, *refs):
        if with_cnt:
            (out_hbm, cnt_hbm, acc_sh, zero_v, srcl, dstl, rows, gsem, ssem,
             cnt_sh, zcnt_v, ones_v, csem) = refs
        else:
            (out_hbm, acc_sh, zero_v, srcl, dstl, rows, gsem, ssem) = refs

        c = lax.axis_index("c")
        s = lax.axis_index("s")
        w = c * _NS + s

        zf = jnp.zeros((_L,), jnp.float32)

        @pl.loop(0, _ZB)
        def _(i):
            for j in range(0, h, _L):
                zero_v[i, pl.ds(j, _L)] = zf

        zbase = s * rps

        @pl.loop(0, nzb)
        def _(k):
            pltpu.sync_copy(zero_v, acc_sh.at[pl.ds(zbase + k * _ZB, _ZB)])

        if with_cnt:
            of = jnp.full((_L,), 1.0, jnp.float32)

            @pl.loop(0, _ZB)
            def _(i):
                zcnt_v[i, pl.ds(0, _L)] = zf

            @pl.loop(0, _CHUNK)
            def _(i):
                ones_v[i, pl.ds(0, _L)] = of

            @pl.loop(0, nzb)
            def _(k):
                pltpu.sync_copy(zcnt_v, cnt_sh.at[pl.ds(zbase + k * _ZB, _ZB)])

        pltpu.sync_copy(src_hbm.at[w], srcl)
        pltpu.sync_copy(dst_hbm.at[w], dstl)

        plsc.subcore_barrier()

        @pl.loop(0, ngroup)
        def _(g):
            j0 = g * _NBUF
            gh = []
            for b in range(_NBUF):
                buf = rows.at[pl.ds(b * _CHUNK, _CHUNK)]
                gh.append(pltpu.async_copy(y_hbm.at[srcl.at[j0 + b]], buf,
                                           gsem.at[b]))
            done = []
            for b in range(_NBUF):
                gh[b].wait()
                buf = rows.at[pl.ds(b * _CHUNK, _CHUNK)]
                done.append(pltpu.async_copy(buf, acc_sh.at[dstl.at[j0 + b]],
                                             ssem.at[b], add=True))
                if with_cnt:
                    done.append(pltpu.async_copy(
                        ones_v, cnt_sh.at[dstl.at[j0 + b]], csem.at[b],
                        add=True))
            for d in done:
                d.wait()

        plsc.subcore_barrier()

        obase = s * rps
        pltpu.sync_copy(acc_sh.at[pl.ds(obase, rps)],
                        out_hbm.at[c].at[pl.ds(obase, rps)])
        if with_cnt:
            pltpu.sync_copy(cnt_sh.at[pl.ds(obase, rps)],
                            cnt_hbm.at[c].at[pl.ds(obase, rps)])

    return sc_kernel


def _proj2(x, wa, wb):
    m = x.shape[0]
    ha = wa.shape[1]
    hb = wb.shape[1]

    def body(x_ref, wa_ref, wb_ref, ya_ref, yb_ref):
        xv = x_ref[...]
        ya_ref[...] = lax.dot(xv, wa_ref[...], precision=_HIGH)
        yb_ref[...] = lax.dot(xv, wb_ref[...], precision=_HIGH)

    return pl.pallas_call(
        body,
        out_shape=(jax.ShapeDtypeStruct((m, ha), jnp.float32),
                   jax.ShapeDtypeStruct((m, hb), jnp.float32)),
    )(x, wa, wb)


def _combine(p_ref, c_ref, r_ref, a_ref, cv_ref, n, h):
    agg = p_ref[0, :n, :] + p_ref[1, :n, :]
    ct = c_ref[0, :n, :] + c_ref[1, :n, :]
    inv = 1.0 / jnp.maximum(ct, 1.0)
    inv_h = jnp.concatenate([inv] * (h // _L), axis=1)
    return jnp.maximum((agg * inv_h + r_ref[...]) * a_ref[...] + cv_ref[...],
                       0.0)


def _mid_layer(p, cnt, r, a, cv, wl, wr, n):
    h = r.shape[1]

    def body(p_ref, c_ref, r_ref, a_ref, cv_ref, wl_ref, wr_ref, y_ref, rr_ref):
        hid = _combine(p_ref, c_ref, r_ref, a_ref, cv_ref, n, h)
        y_ref[...] = lax.dot(hid, wl_ref[...], precision=_HIGH)
        rr_ref[...] = lax.dot(hid, wr_ref[...], precision=_HIGH)

    return pl.pallas_call(
        body,
        out_shape=(jax.ShapeDtypeStruct((n, wl.shape[1]), jnp.float32),
                   jax.ShapeDtypeStruct((n, wr.shape[1]), jnp.float32)),
    )(p, cnt, r, a, cv, wl, wr)


def _final_layer(p, cnt, r, a, cv, wo, bo, n):
    h = r.shape[1]

    def body(p_ref, c_ref, r_ref, a_ref, cv_ref, wo_ref, bo_ref, o_ref):
        hid = _combine(p_ref, c_ref, r_ref, a_ref, cv_ref, n, h)
        o_ref[...] = (jnp.sum(hid * wo_ref[...], axis=1, keepdims=True)
                      + bo_ref[...])

    return pl.pallas_call(
        body,
        out_shape=jax.ShapeDtypeStruct((n, 1), jnp.float32),
    )(p, cnt, r, a, cv, wo, bo)


def kernel(x, edge_index, Wl1, bl1, Wr1, Wl2, bl2, Wr2,
           g1, b1, rm1, rv1, g2, b2, rm2, rv2, Wo, bo):
    n, d = x.shape
    e = edge_index.shape[1]
    h = Wl1.shape[0]

    e_pad = _round_up(e, _NW * _CHUNK * _NBUF)
    n_pad = _round_up(n, _NS * _ZB)
    if e_pad > e and n_pad == n:
        n_pad += _NS * _ZB
    src = edge_index[0]
    dst = edge_index[1]
    if e_pad > e:
        pad = e_pad - e
        src = jnp.concatenate([src, jnp.zeros((pad,), jnp.int32)])
        dst = jnp.concatenate([dst, jnp.full((pad,), n, jnp.int32)])
    nchunk = e_pad // (_NW * _CHUNK)
    src = src.reshape(_NW, nchunk, _CHUNK)
    dst = dst.reshape(_NW, nchunk, _CHUNK)

    a1 = g1 / jnp.sqrt(rv1 + 1e-5)
    c1 = b1 + (bl1 - rm1) * a1
    a2 = g2 / jnp.sqrt(rv2 + 1e-5)
    c2 = b2 + (bl2 - rm2) * a2

    sc1 = _make_sc_segsum(n, h, n_pad, e_pad, with_cnt=True)
    sc2 = _make_sc_segsum(n, h, n_pad, e_pad, with_cnt=False)

    y1, r1 = _proj2(x, Wl1.T, Wr1.T)
    p1, cnt = sc1(y1, src, dst)
    y2, r2 = _mid_layer(p1, cnt, r1, a1.reshape(1, h), c1.reshape(1, h),
                        Wl2.T, Wr2.T, n)
    (p2,) = sc2(y2, src, dst)
    out = _final_layer(p2, cnt, r2, a2.reshape(1, h), c2.reshape(1, h),
                       Wo.reshape(1, h), bo.reshape(1, 1), n)
    return out


# speedup vs baseline: 13.2926x; 2.2107x over previous
import functools

import jax
import jax.numpy as jnp
from jax import lax
from jax.experimental import pallas as pl
from jax.experimental.pallas import tpu as pltpu
from jax.experimental.pallas import tpu_sc as plsc

_NC = 2
_NS = 16
_NW = _NC * _NS
_L = 16
_CHUNK = 128
_ZB = 64

_HIGH = lax.Precision.HIGHEST


def _round_up(a, b):
    return -(-a // b) * b


_NBUF = 5


def _make_sc_segsum(n, h, n_pad, e_pad, with_cnt):
    pw = e_pad // _NW
    nchunk = pw // _CHUNK
    ngroup = nchunk // _NBUF
    rps = n_pad // _NS
    nzb = rps // _ZB
    mesh = plsc.VectorSubcoreMesh(core_axis_name="c", subcore_axis_name="s")

    out_type = [jax.ShapeDtypeStruct((_NC, n_pad, h), jnp.float32)]
    scratch = [
        pltpu.VMEM_SHARED((n_pad, h), jnp.float32),
        pltpu.VMEM((_ZB, h), jnp.float32),
        pltpu.VMEM((nchunk, _CHUNK), jnp.int32),
        pltpu.VMEM((nchunk, _CHUNK), jnp.int32),
        pltpu.VMEM((_NBUF * _CHUNK, h), jnp.float32),
        pltpu.SemaphoreType.DMA((_NBUF,)),
        pltpu.SemaphoreType.DMA((_NBUF,)),
    ]
    if with_cnt:
        out_type.append(jax.ShapeDtypeStruct((_NC, n_pad, _L), jnp.float32))
        scratch += [
            pltpu.VMEM_SHARED((n_pad, _L), jnp.float32),
            pltpu.VMEM((_ZB, _L), jnp.float32),
            pltpu.VMEM((_CHUNK, _L), jnp.float32),
            pltpu.SemaphoreType.DMA((_NBUF,)),
        ]

    @functools.partial(
        pl.kernel, mesh=mesh, out_type=tuple(out_type),
        scratch_types=scratch,
        compiler_params=pltpu.CompilerParams(use_tc_tiling_on_sc=False))
    def sc_kernel(y_hbm, src_hbm, dst_hbm, *refs):
        if with_cnt:
            (out_hbm, cnt_hbm, acc_sh, zero_v, srcl, dstl, rows, gsem, ssem,
             cnt_sh, zcnt_v, ones_v, csem) = refs
        else:
            (out_hbm, acc_sh, zero_v, srcl, dstl, rows, gsem, ssem) = refs

        c = lax.axis_index("c")
        s = lax.axis_index("s")
        w = c * _NS + s

        zf = jnp.zeros((_L,), jnp.float32)

        @pl.loop(0, _ZB)
        def _(i):
            for j in range(0, h, _L):
                zero_v[i, pl.ds(j, _L)] = zf

        zbase = s * rps

        @pl.loop(0, nzb)
        def _(k):
            pltpu.sync_copy(zero_v, acc_sh.at[pl.ds(zbase + k * _ZB, _ZB)])

        if with_cnt:
            of = jnp.full((_L,), 1.0, jnp.float32)

            @pl.loop(0, _ZB)
            def _(i):
                zcnt_v[i, pl.ds(0, _L)] = zf

            @pl.loop(0, _CHUNK)
            def _(i):
                ones_v[i, pl.ds(0, _L)] = of

            @pl.loop(0, nzb)
            def _(k):
                pltpu.sync_copy(zcnt_v, cnt_sh.at[pl.ds(zbase + k * _ZB, _ZB)])

        pltpu.sync_copy(src_hbm.at[w], srcl)
        pltpu.sync_copy(dst_hbm.at[w], dstl)

        plsc.subcore_barrier()

        @pl.loop(0, ngroup)
        def _(g):
            j0 = g * _NBUF
            gh = []
            for b in range(_NBUF):
                buf = rows.at[pl.ds(b * _CHUNK, _CHUNK)]
                gh.append(pltpu.async_copy(y_hbm.at[srcl.at[j0 + b]], buf,
                                           gsem.at[b]))
            done = []
            for b in range(_NBUF):
                gh[b].wait()
                buf = rows.at[pl.ds(b * _CHUNK, _CHUNK)]
                done.append(pltpu.async_copy(buf, acc_sh.at[dstl.at[j0 + b]],
                                             ssem.at[b], add=True))
                if with_cnt:
                    done.append(pltpu.async_copy(
                        ones_v, cnt_sh.at[dstl.at[j0 + b]], csem.at[b],
                        add=True))
            for d in done:
                d.wait()

        plsc.subcore_barrier()

        obase = s * rps
        pltpu.sync_copy(acc_sh.at[pl.ds(obase, rps)],
                        out_hbm.at[c].at[pl.ds(obase, rps)])
        if with_cnt:
            pltpu.sync_copy(cnt_sh.at[pl.ds(obase, rps)],
                            cnt_hbm.at[c].at[pl.ds(obase, rps)])

    return sc_kernel


def _proj2(x, wa, wb):
    m = x.shape[0]
    ha = wa.shape[1]
    hb = wb.shape[1]

    def body(x_ref, wa_ref, wb_ref, ya_ref, yb_ref):
        xv = x_ref[...]
        ya_ref[...] = lax.dot(xv, wa_ref[...], precision=_HIGH)
        yb_ref[...] = lax.dot(xv, wb_ref[...], precision=_HIGH)

    return pl.pallas_call(
        body,
        out_shape=(jax.ShapeDtypeStruct((m, ha), jnp.float32),
                   jax.ShapeDtypeStruct((m, hb), jnp.float32)),
    )(x, wa, wb)


def _combine(p_ref, c_ref, r_ref, a_ref, cv_ref, n, h):
    agg = p_ref[0, :n, :] + p_ref[1, :n, :]
    ct = c_ref[0, :n, :] + c_ref[1, :n, :]
    inv = 1.0 / jnp.maximum(ct, 1.0)
    inv_h = jnp.concatenate([inv] * (h // _L), axis=1)
    return jnp.maximum((agg * inv_h + r_ref[...]) * a_ref[...] + cv_ref[...],
                       0.0)


def _mid_layer(p, cnt, r, a, cv, wl, wr, n):
    h = r.shape[1]

    def body(p_ref, c_ref, r_ref, a_ref, cv_ref, wl_ref, wr_ref, y_ref, rr_ref):
        hid = _combine(p_ref, c_ref, r_ref, a_ref, cv_ref, n, h)
        y_ref[...] = lax.dot(hid, wl_ref[...], precision=_HIGH)
        rr_ref[...] = lax.dot(hid, wr_ref[...], precision=_HIGH)

    return pl.pallas_call(
        body,
        out_shape=(jax.ShapeDtypeStruct((n, wl.shape[1]), jnp.float32),
                   jax.ShapeDtypeStruct((n, wr.shape[1]), jnp.float32)),
    )(p, cnt, r, a, cv, wl, wr)


def _final_layer(p, cnt, r, a, cv, wo, bo, n):
    h = r.shape[1]

    def body(p_ref, c_ref, r_ref, a_ref, cv_ref, wo_ref, bo_ref, o_ref):
        hid = _combine(p_ref, c_ref, r_ref, a_ref, cv_ref, n, h)
        o_ref[...] = (jnp.sum(hid * wo_ref[...], axis=1, keepdims=True)
                      + bo_ref[...])

    return pl.pallas_call(
        body,
        out_shape=jax.ShapeDtypeStruct((n, 1), jnp.float32),
    )(p, cnt, r, a, cv, wo, bo)


def kernel(x, edge_index, Wl1, bl1, Wr1, Wl2, bl2, Wr2,
           g1, b1, rm1, rv1, g2, b2, rm2, rv2, Wo, bo):
    n, d = x.shape
    e = edge_index.shape[1]
    h = Wl1.shape[0]

    e_pad = _round_up(e, _NW * _CHUNK * _NBUF)
    n_pad = _round_up(n, _NS * _ZB)
    if e_pad > e and n_pad == n:
        n_pad += _NS * _ZB
    src = edge_index[0]
    dst = edge_index[1]
    if e_pad > e:
        pad = e_pad - e
        ar = jnp.arange(pad, dtype=jnp.int32)
        src = jnp.concatenate([src, ar % n])
        dst = jnp.concatenate([dst, n + ar % (n_pad - n)])
    nchunk = e_pad // (_NW * _CHUNK)
    src = src.reshape(_NW, nchunk, _CHUNK)
    dst = dst.reshape(_NW, nchunk, _CHUNK)

    a1 = g1 / jnp.sqrt(rv1 + 1e-5)
    c1 = b1 + (bl1 - rm1) * a1
    a2 = g2 / jnp.sqrt(rv2 + 1e-5)
    c2 = b2 + (bl2 - rm2) * a2

    sc1 = _make_sc_segsum(n, h, n_pad, e_pad, with_cnt=True)
    sc2 = _make_sc_segsum(n, h, n_pad, e_pad, with_cnt=False)

    y1, r1 = _proj2(x, Wl1.T, Wr1.T)
    p1, cnt = sc1(y1, src, dst)
    y2, r2 = _mid_layer(p1, cnt, r1, a1.reshape(1, h), c1.reshape(1, h),
                        Wl2.T, Wr2.T, n)
    (p2,) = sc2(y2, src, dst)
    out = _final_layer(p2, cnt, r2, a2.reshape(1, h), c2.reshape(1, h),
                       Wo.reshape(1, h), bo.reshape(1, 1), n)
    return out


# speedup vs baseline: 14.9600x; 1.1254x over previous
import functools

import jax
import jax.numpy as jnp
from jax import lax
from jax.scipy.linalg import block_diag
from jax.experimental import pallas as pl
from jax.experimental.pallas import tpu as pltpu
from jax.experimental.pallas import tpu_sc as plsc

_NC = 2
_NS = 16
_NW = _NC * _NS
_L = 16
_CHUNK = 128
_ZB = 64

_HIGH = lax.Precision.HIGHEST


def _round_up(a, b):
    return -(-a // b) * b


_NBUF = 5


def _make_sc_segsum(n, h, n_pad, e_pad, with_cnt):
    pw = e_pad // _NW
    nchunk = pw // _CHUNK
    ngroup = nchunk // _NBUF
    rps = n_pad // _NS
    nzb = rps // _ZB
    mesh = plsc.VectorSubcoreMesh(core_axis_name="c", subcore_axis_name="s")

    out_type = [jax.ShapeDtypeStruct((_NC, n_pad, h), jnp.float32)]
    scratch = [
        pltpu.VMEM_SHARED((n_pad, h), jnp.float32),
        pltpu.VMEM((_ZB, h), jnp.float32),
        pltpu.VMEM((nchunk, _CHUNK), jnp.int32),
        pltpu.VMEM((nchunk, _CHUNK), jnp.int32),
        pltpu.VMEM((_NBUF * _CHUNK, h), jnp.float32),
        pltpu.SemaphoreType.DMA((_NBUF,)),
        pltpu.SemaphoreType.DMA((_NBUF,)),
    ]
    if with_cnt:
        out_type.append(jax.ShapeDtypeStruct((_NC, n_pad, _L), jnp.float32))
        scratch += [
            pltpu.VMEM_SHARED((n_pad, _L), jnp.float32),
            pltpu.VMEM((_ZB, _L), jnp.float32),
            pltpu.VMEM((_CHUNK, _L), jnp.float32),
            pltpu.SemaphoreType.DMA((_NBUF,)),
        ]

    @functools.partial(
        pl.kernel, mesh=mesh, out_type=tuple(out_type),
        scratch_types=scratch,
        compiler_params=pltpu.CompilerParams(use_tc_tiling_on_sc=False))
    def sc_kernel(y_hbm, src_hbm, dst_hbm, *refs):
        if with_cnt:
            (out_hbm, cnt_hbm, acc_sh, zero_v, srcl, dstl, rows, gsem, ssem,
             cnt_sh, zcnt_v, ones_v, csem) = refs
        else:
            (out_hbm, acc_sh, zero_v, srcl, dstl, rows, gsem, ssem) = refs

        c = lax.axis_index("c")
        s = lax.axis_index("s")
        w = c * _NS + s

        zf = jnp.zeros((_L,), jnp.float32)

        @pl.loop(0, _ZB)
        def _(i):
            for j in range(0, h, _L):
                zero_v[i, pl.ds(j, _L)] = zf

        zbase = s * rps

        @pl.loop(0, nzb)
        def _(k):
            pltpu.sync_copy(zero_v, acc_sh.at[pl.ds(zbase + k * _ZB, _ZB)])

        if with_cnt:
            of = jnp.full((_L,), 1.0, jnp.float32)

            @pl.loop(0, _ZB)
            def _(i):
                zcnt_v[i, pl.ds(0, _L)] = zf

            @pl.loop(0, _CHUNK)
            def _(i):
                ones_v[i, pl.ds(0, _L)] = of

            @pl.loop(0, nzb)
            def _(k):
                pltpu.sync_copy(zcnt_v, cnt_sh.at[pl.ds(zbase + k * _ZB, _ZB)])

        pltpu.sync_copy(src_hbm.at[w], srcl)
        pltpu.sync_copy(dst_hbm.at[w], dstl)

        plsc.subcore_barrier()

        @pl.loop(0, ngroup)
        def _(g):
            j0 = g * _NBUF
            gh = []
            for b in range(_NBUF):
                buf = rows.at[pl.ds(b * _CHUNK, _CHUNK)]
                gh.append(pltpu.async_copy(y_hbm.at[srcl.at[j0 + b]], buf,
                                           gsem.at[b]))
            done = []
            for b in range(_NBUF):
                gh[b].wait()
                buf = rows.at[pl.ds(b * _CHUNK, _CHUNK)]
                done.append(pltpu.async_copy(buf, acc_sh.at[dstl.at[j0 + b]],
                                             ssem.at[b], add=True))
                if with_cnt:
                    done.append(pltpu.async_copy(
                        ones_v, cnt_sh.at[dstl.at[j0 + b]], csem.at[b],
                        add=True))
            for d in done:
                d.wait()

        plsc.subcore_barrier()

        obase = s * rps
        pltpu.sync_copy(acc_sh.at[pl.ds(obase, rps)],
                        out_hbm.at[c].at[pl.ds(obase, rps)])
        if with_cnt:
            pltpu.sync_copy(cnt_sh.at[pl.ds(obase, rps)],
                            cnt_hbm.at[c].at[pl.ds(obase, rps)])

    return sc_kernel


_BR = 320


def _grid(m):
    return (-(-m // _BR),)


def _proj2(x, wa, wb):
    m = x.shape[0]
    d2 = x.shape[1]
    ha = wa.shape[1]
    hb = wb.shape[1]

    def body(x_ref, wa_ref, wb_ref, ya_ref, yb_ref):
        xv = x_ref[...]
        ya_ref[...] = lax.dot(xv, wa_ref[...], precision=_HIGH)
        yb_ref[...] = lax.dot(xv, wb_ref[...], precision=_HIGH)

    return pl.pallas_call(
        body,
        grid=_grid(m),
        in_specs=[
            pl.BlockSpec((_BR, d2), lambda i: (i, 0)),
            pl.BlockSpec(wa.shape, lambda i: (0, 0)),
            pl.BlockSpec(wb.shape, lambda i: (0, 0)),
        ],
        out_specs=(pl.BlockSpec((_BR, ha), lambda i: (i, 0)),
                   pl.BlockSpec((_BR, hb), lambda i: (i, 0))),
        out_shape=(jax.ShapeDtypeStruct((m, ha), jnp.float32),
                   jax.ShapeDtypeStruct((m, hb), jnp.float32)),
    )(x, wa, wb)


def _combine(p_ref, c_ref, r_ref, a_ref, cv_ref, m, qn):
    agg = p_ref[0, :m, :] + p_ref[1, :m, :]
    ct = c_ref[0, :qn, :] + c_ref[1, :qn, :]
    iv = 1.0 / jnp.maximum(ct, 1.0)
    pieces = [
        jnp.concatenate([iv[:, 32 * j:32 * j + 16]] * 4
                        + [iv[:, 32 * j + 16:32 * j + 32]] * 4, axis=1)
        for j in range(4)
    ]
    inv = jnp.stack(pieces, axis=1).reshape(m, 128)
    return jnp.maximum((agg * inv + r_ref[...]) * a_ref[...] + cv_ref[...],
                       0.0)


def _node_specs(extra):
    return [
        pl.BlockSpec((_NC, _BR, 128), lambda i: (0, i, 0)),
        pl.BlockSpec((_NC, _BR // 4, 128), lambda i: (0, i, 0)),
        pl.BlockSpec((_BR, 128), lambda i: (i, 0)),
        pl.BlockSpec((1, 128), lambda i: (0, 0)),
        pl.BlockSpec((1, 128), lambda i: (0, 0)),
    ] + extra


def _mid_layer(p, cnt, r, a, cv, wl, wr):
    m = r.shape[0]

    def body(p_ref, c_ref, r_ref, a_ref, cv_ref, wl_ref, wr_ref, y_ref, rr_ref):
        hid = _combine(p_ref, c_ref, r_ref, a_ref, cv_ref, _BR, _BR // 4)
        y_ref[...] = lax.dot(hid, wl_ref[...], precision=_HIGH)
        rr_ref[...] = lax.dot(hid, wr_ref[...], precision=_HIGH)

    return pl.pallas_call(
        body,
        grid=_grid(m),
        in_specs=_node_specs([pl.BlockSpec(wl.shape, lambda i: (0, 0)),
                              pl.BlockSpec(wr.shape, lambda i: (0, 0))]),
        out_specs=(pl.BlockSpec((_BR, 128), lambda i: (i, 0)),
                   pl.BlockSpec((_BR, 128), lambda i: (i, 0))),
        out_shape=(jax.ShapeDtypeStruct((m, 128), jnp.float32),
                   jax.ShapeDtypeStruct((m, 128), jnp.float32)),
    )(p, cnt, r, a, cv, wl, wr)


def _final_layer(p, cnt, r, a, cv, wo, bo):
    m = r.shape[0]

    def body(p_ref, c_ref, r_ref, a_ref, cv_ref, wo_ref, bo_ref, o_ref):
        hid = _combine(p_ref, c_ref, r_ref, a_ref, cv_ref, _BR, _BR // 4)
        t = hid * wo_ref[...]
        lo = jnp.sum(t[:, :64], axis=1, keepdims=True)
        hi = jnp.sum(t[:, 64:], axis=1, keepdims=True)
        o_ref[...] = jnp.concatenate([lo, hi], axis=1) + bo_ref[...]

    return pl.pallas_call(
        body,
        grid=_grid(m),
        in_specs=_node_specs([pl.BlockSpec((1, 128), lambda i: (0, 0)),
                              pl.BlockSpec((1, 1), lambda i: (0, 0))]),
        out_specs=pl.BlockSpec((_BR, 2), lambda i: (i, 0)),
        out_shape=jax.ShapeDtypeStruct((m, 2), jnp.float32),
    )(p, cnt, r, a, cv, wo, bo)


def kernel(x, edge_index, Wl1, bl1, Wr1, Wl2, bl2, Wr2,
           g1, b1, rm1, rv1, g2, b2, rm2, rv2, Wo, bo):
    n, d = x.shape
    e = edge_index.shape[1]
    h = Wl1.shape[0]

    e_pad = _round_up(e, _NW * _CHUNK * _NBUF)
    n_pad = _round_up(n, _NS * _ZB)
    if e_pad > e and n_pad == n:
        n_pad += _NS * _ZB
    src = edge_index[0]
    dst = edge_index[1]
    if e_pad > e:
        pad = e_pad - e
        ar = jnp.arange(pad, dtype=jnp.int32)
        src = jnp.concatenate([src, ar % n])
        dst = jnp.concatenate([dst, n + ar % (n_pad - n)])
    nchunk = e_pad // (_NW * _CHUNK)
    src = src.reshape(_NW, nchunk, _CHUNK)
    dst = dst.reshape(_NW, nchunk, _CHUNK)

    a1 = g1 / jnp.sqrt(rv1 + 1e-5)
    c1 = b1 + (bl1 - rm1) * a1
    a2 = g2 / jnp.sqrt(rv2 + 1e-5)
    c2 = b2 + (bl2 - rm2) * a2

    sc1 = _make_sc_segsum(n, h, n_pad, e_pad, with_cnt=True)
    sc2 = _make_sc_segsum(n, h, n_pad, e_pad, with_cnt=False)

    bd = block_diag
    two = lambda v: jnp.concatenate([v, v]).reshape(1, 2 * h)
    x_pair = x.reshape(n // 2, 2 * d)

    y1p, r1p = _proj2(x_pair, bd(Wl1.T, Wl1.T), bd(Wr1.T, Wr1.T))
    p1, cnt = sc1(y1p.reshape(n, h), src, dst)
    p1p = p1.reshape(_NC, n_pad // 2, 128)
    cntp = cnt.reshape(_NC, n_pad * _L // 128, 128)
    y2p, r2p = _mid_layer(p1p, cntp, r1p, two(a1), two(c1),
                          bd(Wl2.T, Wl2.T), bd(Wr2.T, Wr2.T))
    (p2,) = sc2(y2p.reshape(n, h), src, dst)
    p2p = p2.reshape(_NC, n_pad // 2, 128)
    out = _final_layer(p2p, cntp, r2p, two(a2), two(c2),
                       two(Wo.reshape(h)), bo.reshape(1, 1))
    return out.reshape(n, 1)


# speedup vs baseline: 18.8833x; 1.2622x over previous
import functools

import jax
import jax.numpy as jnp
from jax import lax
from jax.scipy.linalg import block_diag
from jax.experimental import pallas as pl
from jax.experimental.pallas import tpu as pltpu
from jax.experimental.pallas import tpu_sc as plsc

_NC = 2
_NS = 16
_NW = _NC * _NS
_L = 16
_CHUNK = 128
_ZB = 64

_HIGH = lax.Precision.HIGHEST


def _round_up(a, b):
    return -(-a // b) * b


_NBUF = 5


def _make_sc_segsum(n, h, n_pad, e_pad, with_cnt):
    pw = e_pad // _NW
    nchunk = pw // _CHUNK
    ngroup = nchunk // _NBUF
    rps = n_pad // _NS
    nzb = rps // _ZB
    mesh = plsc.VectorSubcoreMesh(core_axis_name="c", subcore_axis_name="s")

    out_type = [jax.ShapeDtypeStruct((_NC, n_pad, h), jnp.bfloat16)]
    scratch = [
        pltpu.VMEM_SHARED((n_pad, h), jnp.bfloat16),
        pltpu.VMEM((_ZB, h), jnp.bfloat16),
        pltpu.VMEM((nchunk, _CHUNK), jnp.int32),
        pltpu.VMEM((nchunk, _CHUNK), jnp.int32),
        pltpu.VMEM((_NBUF * _CHUNK, h), jnp.bfloat16),
        pltpu.SemaphoreType.DMA((_NBUF,)),
        pltpu.SemaphoreType.DMA((_NBUF,)),
    ]
    if with_cnt:
        out_type.append(jax.ShapeDtypeStruct((_NC, n_pad, _L), jnp.float32))
        scratch += [
            pltpu.VMEM_SHARED((n_pad, _L), jnp.float32),
            pltpu.VMEM((_ZB, _L), jnp.float32),
            pltpu.VMEM((_CHUNK, _L), jnp.float32),
            pltpu.SemaphoreType.DMA((_NBUF,)),
        ]

    @functools.partial(
        pl.kernel, mesh=mesh, out_type=tuple(out_type),
        scratch_types=scratch,
        compiler_params=pltpu.CompilerParams(use_tc_tiling_on_sc=False))
    def sc_kernel(y_hbm, src_hbm, dst_hbm, *refs):
        if with_cnt:
            (out_hbm, cnt_hbm, acc_sh, zero_v, srcl, dstl, rows, gsem, ssem,
             cnt_sh, zcnt_v, ones_v, csem) = refs
        else:
            (out_hbm, acc_sh, zero_v, srcl, dstl, rows, gsem, ssem) = refs

        c = lax.axis_index("c")
        s = lax.axis_index("s")
        w = c * _NS + s

        zb = jnp.zeros((2 * _L,), jnp.bfloat16)

        @pl.loop(0, _ZB)
        def _(i):
            for j in range(0, h, 2 * _L):
                zero_v[i, pl.ds(j, 2 * _L)] = zb

        zbase = s * rps

        @pl.loop(0, nzb)
        def _(k):
            pltpu.sync_copy(zero_v, acc_sh.at[pl.ds(zbase + k * _ZB, _ZB)])

        if with_cnt:
            of = jnp.full((_L,), 1.0, jnp.float32)
            zf = jnp.zeros((_L,), jnp.float32)

            @pl.loop(0, _ZB)
            def _(i):
                zcnt_v[i, pl.ds(0, _L)] = zf

            @pl.loop(0, _CHUNK)
            def _(i):
                ones_v[i, pl.ds(0, _L)] = of

            @pl.loop(0, nzb)
            def _(k):
                pltpu.sync_copy(zcnt_v, cnt_sh.at[pl.ds(zbase + k * _ZB, _ZB)])

        pltpu.sync_copy(src_hbm.at[w], srcl)
        pltpu.sync_copy(dst_hbm.at[w], dstl)

        plsc.subcore_barrier()

        def buf(b):
            return rows.at[pl.ds(b * _CHUNK, _CHUNK)]

        def wait_gather(b):
            pltpu.make_async_copy(y_hbm.at[pl.ds(0, _CHUNK)], buf(b),
                                  gsem.at[b]).wait()

        def wait_scatter(b):
            pltpu.make_async_copy(y_hbm.at[pl.ds(0, _CHUNK)], buf(b),
                                  ssem.at[b]).wait()
            if with_cnt:
                pltpu.make_async_copy(cnt_hbm.at[0].at[pl.ds(0, _CHUNK)],
                                      ones_v, csem.at[b]).wait()

        for b in range(_NBUF):
            pltpu.async_copy(y_hbm.at[srcl.at[b]], buf(b), gsem.at[b])

        @pl.loop(0, ngroup)
        def _(g):
            j0 = g * _NBUF
            jn = jnp.minimum(g + 1, ngroup - 1) * _NBUF
            for b in range(_NBUF):
                wait_gather(b)
                pltpu.async_copy(buf(b), acc_sh.at[dstl.at[j0 + b]],
                                 ssem.at[b], add=True)
                if with_cnt:
                    pltpu.async_copy(ones_v, cnt_sh.at[dstl.at[j0 + b]],
                                     csem.at[b], add=True)
            for b in range(_NBUF):
                wait_scatter(b)
                pltpu.async_copy(y_hbm.at[srcl.at[jn + b]], buf(b),
                                 gsem.at[b])

        for b in range(_NBUF):
            wait_gather(b)

        plsc.subcore_barrier()

        obase = s * rps
        pltpu.sync_copy(acc_sh.at[pl.ds(obase, rps)],
                        out_hbm.at[c].at[pl.ds(obase, rps)])
        if with_cnt:
            pltpu.sync_copy(cnt_sh.at[pl.ds(obase, rps)],
                            cnt_hbm.at[c].at[pl.ds(obase, rps)])

    return sc_kernel


_BR = 320


def _grid(m):
    return (-(-m // _BR),)


def _proj2(x, wa, wb):
    m = x.shape[0]
    d2 = x.shape[1]
    ha = wa.shape[1]
    hb = wb.shape[1]

    def body(x_ref, wa_ref, wb_ref, ya_ref, yb_ref):
        xv = x_ref[...]
        ya_ref[...] = lax.dot(xv, wa_ref[...],
                              precision=_HIGH).astype(jnp.bfloat16)
        yb_ref[...] = lax.dot(xv, wb_ref[...], precision=_HIGH)

    return pl.pallas_call(
        body,
        grid=_grid(m),
        in_specs=[
            pl.BlockSpec((_BR, d2), lambda i: (i, 0)),
            pl.BlockSpec(wa.shape, lambda i: (0, 0)),
            pl.BlockSpec(wb.shape, lambda i: (0, 0)),
        ],
        out_specs=(pl.BlockSpec((_BR, ha), lambda i: (i, 0)),
                   pl.BlockSpec((_BR, hb), lambda i: (i, 0))),
        out_shape=(jax.ShapeDtypeStruct((m, ha), jnp.bfloat16),
                   jax.ShapeDtypeStruct((m, hb), jnp.float32)),
    )(x, wa, wb)


def _combine(p_ref, c_ref, r_ref, a_ref, cv_ref, m, qn):
    agg = (p_ref[0, :m, :].astype(jnp.float32)
           + p_ref[1, :m, :].astype(jnp.float32))
    ct = c_ref[0, :qn, :] + c_ref[1, :qn, :]
    iv = 1.0 / jnp.maximum(ct, 1.0)
    pieces = [
        jnp.concatenate([iv[:, 32 * j:32 * j + 16]] * 4
                        + [iv[:, 32 * j + 16:32 * j + 32]] * 4, axis=1)
        for j in range(4)
    ]
    inv = jnp.stack(pieces, axis=1).reshape(m, 128)
    return jnp.maximum((agg * inv + r_ref[...]) * a_ref[...] + cv_ref[...],
                       0.0)


def _node_specs(extra):
    return [
        pl.BlockSpec((_NC, _BR, 128), lambda i: (0, i, 0)),
        pl.BlockSpec((_NC, _BR // 4, 128), lambda i: (0, i, 0)),
        pl.BlockSpec((_BR, 128), lambda i: (i, 0)),
        pl.BlockSpec((1, 128), lambda i: (0, 0)),
        pl.BlockSpec((1, 128), lambda i: (0, 0)),
    ] + extra


def _mid_layer(p, cnt, r, a, cv, wl, wr):
    m = r.shape[0]

    def body(p_ref, c_ref, r_ref, a_ref, cv_ref, wl_ref, wr_ref, y_ref, rr_ref):
        hid = _combine(p_ref, c_ref, r_ref, a_ref, cv_ref, _BR, _BR // 4)
        y_ref[...] = lax.dot(hid, wl_ref[...],
                             precision=_HIGH).astype(jnp.bfloat16)
        rr_ref[...] = lax.dot(hid, wr_ref[...], precision=_HIGH)

    return pl.pallas_call(
        body,
        grid=_grid(m),
        in_specs=_node_specs([pl.BlockSpec(wl.shape, lambda i: (0, 0)),
                              pl.BlockSpec(wr.shape, lambda i: (0, 0))]),
        out_specs=(pl.BlockSpec((_BR, 128), lambda i: (i, 0)),
                   pl.BlockSpec((_BR, 128), lambda i: (i, 0))),
        out_shape=(jax.ShapeDtypeStruct((m, 128), jnp.bfloat16),
                   jax.ShapeDtypeStruct((m, 128), jnp.float32)),
    )(p, cnt, r, a, cv, wl, wr)


def _final_layer(p, cnt, r, a, cv, wo, bo):
    m = r.shape[0]

    def body(p_ref, c_ref, r_ref, a_ref, cv_ref, wo_ref, bo_ref, o_ref):
        hid = _combine(p_ref, c_ref, r_ref, a_ref, cv_ref, _BR, _BR // 4)
        t = hid * wo_ref[...]
        lo = jnp.sum(t[:, :64], axis=1, keepdims=True)
        hi = jnp.sum(t[:, 64:], axis=1, keepdims=True)
        o_ref[...] = jnp.concatenate([lo, hi], axis=1) + bo_ref[...]

    return pl.pallas_call(
        body,
        grid=_grid(m),
        in_specs=_node_specs([pl.BlockSpec((1, 128), lambda i: (0, 0)),
                              pl.BlockSpec((1, 1), lambda i: (0, 0))]),
        out_specs=pl.BlockSpec((_BR, 2), lambda i: (i, 0)),
        out_shape=jax.ShapeDtypeStruct((m, 2), jnp.float32),
    )(p, cnt, r, a, cv, wo, bo)


def kernel(x, edge_index, Wl1, bl1, Wr1, Wl2, bl2, Wr2,
           g1, b1, rm1, rv1, g2, b2, rm2, rv2, Wo, bo):
    n, d = x.shape
    e = edge_index.shape[1]
    h = Wl1.shape[0]

    e_pad = _round_up(e, _NW * _CHUNK * _NBUF)
    n_pad = _round_up(n, _NS * _ZB)
    if e_pad > e and n_pad == n:
        n_pad += _NS * _ZB
    src = edge_index[0]
    dst = edge_index[1]
    if e_pad > e:
        pad = e_pad - e
        ar = jnp.arange(pad, dtype=jnp.int32)
        src = jnp.concatenate([src, ar % n])
        dst = jnp.concatenate([dst, n + ar % (n_pad - n)])
    nchunk = e_pad // (_NW * _CHUNK)
    src = src.reshape(_NW, nchunk, _CHUNK)
    dst = dst.reshape(_NW, nchunk, _CHUNK)

    a1 = g1 / jnp.sqrt(rv1 + 1e-5)
    c1 = b1 + (bl1 - rm1) * a1
    a2 = g2 / jnp.sqrt(rv2 + 1e-5)
    c2 = b2 + (bl2 - rm2) * a2

    sc1 = _make_sc_segsum(n, h, n_pad, e_pad, with_cnt=True)
    sc2 = _make_sc_segsum(n, h, n_pad, e_pad, with_cnt=False)

    bd = block_diag
    two = lambda v: jnp.concatenate([v, v]).reshape(1, 2 * h)
    x_pair = x.reshape(n // 2, 2 * d)

    y1p, r1p = _proj2(x_pair, bd(Wl1.T, Wl1.T), bd(Wr1.T, Wr1.T))
    p1, cnt = sc1(y1p.reshape(n, h), src, dst)
    p1p = p1.reshape(_NC, n_pad // 2, 128)
    cntp = cnt.reshape(_NC, n_pad * _L // 128, 128)
    y2p, r2p = _mid_layer(p1p, cntp, r1p, two(a1), two(c1),
                          bd(Wl2.T, Wl2.T), bd(Wr2.T, Wr2.T))
    (p2,) = sc2(y2p.reshape(n, h), src, dst)
    p2p = p2.reshape(_NC, n_pad // 2, 128)
    out = _final_layer(p2p, cntp, r2p, two(a2), two(c2),
                       two(Wo.reshape(h)), bo.reshape(1, 1))
    return out.reshape(n, 1)


# speedup vs baseline: 18.9859x; 1.0054x over previous
import functools

import jax
import jax.numpy as jnp
from jax import lax
from jax.scipy.linalg import block_diag
from jax.experimental import pallas as pl
from jax.experimental.pallas import tpu as pltpu
from jax.experimental.pallas import tpu_sc as plsc

_NC = 2
_NS = 16
_NW = _NC * _NS
_L = 16
_CHUNK = 128
_ZB = 64

_HIGH = lax.Precision.HIGHEST


def _round_up(a, b):
    return -(-a // b) * b


_NBUF = 5


def _make_sc_segsum(n, h, n_pad, e_pad, with_cnt):
    pw = e_pad // _NW
    nchunk = pw // _CHUNK
    ngroup = nchunk // _NBUF
    rps = n_pad // _NS
    nzb = rps // _ZB
    mesh = plsc.VectorSubcoreMesh(core_axis_name="c", subcore_axis_name="s")

    out_type = [jax.ShapeDtypeStruct((_NC, n_pad, h), jnp.float32)]
    scratch = [
        pltpu.VMEM_SHARED((n_pad, h), jnp.float32),
        pltpu.VMEM((_ZB, h), jnp.float32),
        pltpu.VMEM((nchunk, _CHUNK), jnp.int32),
        pltpu.VMEM((nchunk, _CHUNK), jnp.int32),
        pltpu.VMEM((_NBUF * _CHUNK, h), jnp.float32),
        pltpu.SemaphoreType.DMA((_NBUF,)),
        pltpu.SemaphoreType.DMA((_NBUF,)),
    ]
    if with_cnt:
        out_type.append(jax.ShapeDtypeStruct((_NC, n_pad, _L), jnp.float32))
        scratch += [
            pltpu.VMEM_SHARED((n_pad, _L), jnp.float32),
            pltpu.VMEM((_ZB, _L), jnp.float32),
            pltpu.VMEM((_CHUNK, _L), jnp.float32),
            pltpu.SemaphoreType.DMA((_NBUF,)),
        ]

    @functools.partial(
        pl.kernel, mesh=mesh, out_type=tuple(out_type),
        scratch_types=scratch,
        compiler_params=pltpu.CompilerParams(use_tc_tiling_on_sc=False))
    def sc_kernel(y_hbm, src_hbm, dst_hbm, *refs):
        if with_cnt:
            (out_hbm, cnt_hbm, acc_sh, zero_v, srcl, dstl, rows, gsem, ssem,
             cnt_sh, zcnt_v, ones_v, csem) = refs
        else:
            (out_hbm, acc_sh, zero_v, srcl, dstl, rows, gsem, ssem) = refs

        c = lax.axis_index("c")
        s = lax.axis_index("s")
        w = c * _NS + s

        zf = jnp.zeros((_L,), jnp.float32)

        @pl.loop(0, _ZB)
        def _(i):
            for j in range(0, h, _L):
                zero_v[i, pl.ds(j, _L)] = zf

        zbase = s * rps

        @pl.loop(0, nzb)
        def _(k):
            pltpu.sync_copy(zero_v, acc_sh.at[pl.ds(zbase + k * _ZB, _ZB)])

        if with_cnt:
            of = jnp.full((_L,), 1.0, jnp.float32)

            @pl.loop(0, _ZB)
            def _(i):
                zcnt_v[i, pl.ds(0, _L)] = zf

            @pl.loop(0, _CHUNK)
            def _(i):
                ones_v[i, pl.ds(0, _L)] = of

            @pl.loop(0, nzb)
            def _(k):
                pltpu.sync_copy(zcnt_v, cnt_sh.at[pl.ds(zbase + k * _ZB, _ZB)])

        pltpu.sync_copy(src_hbm.at[w], srcl)
        pltpu.sync_copy(dst_hbm.at[w], dstl)

        plsc.subcore_barrier()

        def buf(b):
            return rows.at[pl.ds(b * _CHUNK, _CHUNK)]

        def wait_gather(b):
            pltpu.make_async_copy(y_hbm.at[pl.ds(0, _CHUNK)], buf(b),
                                  gsem.at[b]).wait()

        def wait_scatter(b):
            pltpu.make_async_copy(y_hbm.at[pl.ds(0, _CHUNK)], buf(b),
                                  ssem.at[b]).wait()
            if with_cnt:
                pltpu.make_async_copy(cnt_hbm.at[0].at[pl.ds(0, _CHUNK)],
                                      ones_v, csem.at[b]).wait()

        for b in range(_NBUF):
            pltpu.async_copy(y_hbm.at[srcl.at[b]], buf(b), gsem.at[b])

        @pl.loop(0, ngroup)
        def _(g):
            j0 = g * _NBUF
            jn = jnp.minimum(g + 1, ngroup - 1) * _NBUF
            for b in range(_NBUF):
                wait_gather(b)
                pltpu.async_copy(buf(b), acc_sh.at[dstl.at[j0 + b]],
                                 ssem.at[b], add=True)
                if with_cnt:
                    pltpu.async_copy(ones_v, cnt_sh.at[dstl.at[j0 + b]],
                                     csem.at[b], add=True)
            for b in range(_NBUF):
                wait_scatter(b)
                pltpu.async_copy(y_hbm.at[srcl.at[jn + b]], buf(b),
                                 gsem.at[b])

        for b in range(_NBUF):
            wait_gather(b)

        plsc.subcore_barrier()

        obase = s * rps
        pltpu.sync_copy(acc_sh.at[pl.ds(obase, rps)],
                        out_hbm.at[c].at[pl.ds(obase, rps)])
        if with_cnt:
            pltpu.sync_copy(cnt_sh.at[pl.ds(obase, rps)],
                            cnt_hbm.at[c].at[pl.ds(obase, rps)])

    return sc_kernel


_BR = 1280


def _grid(m):
    return (-(-m // _BR),)


def _proj1(x, wa):
    m = x.shape[0]
    d2 = x.shape[1]
    ha = wa.shape[1]

    def body(x_ref, wa_ref, ya_ref):
        ya_ref[...] = lax.dot(x_ref[...], wa_ref[...], precision=_HIGH)

    return pl.pallas_call(
        body,
        grid=_grid(m),
        in_specs=[
            pl.BlockSpec((_BR, d2), lambda i: (i, 0)),
            pl.BlockSpec(wa.shape, lambda i: (0, 0)),
        ],
        out_specs=pl.BlockSpec((_BR, ha), lambda i: (i, 0)),
        out_shape=jax.ShapeDtypeStruct((m, ha), jnp.float32),
    )(x, wa)


def _combine(p_ref, c_ref, r_ref, a_ref, cv_ref, m, qn):
    agg = p_ref[0, :m, :] + p_ref[1, :m, :]
    ct = c_ref[0, :qn, :] + c_ref[1, :qn, :]
    iv = 1.0 / jnp.maximum(ct, 1.0)
    pieces = [
        jnp.concatenate([iv[:, 32 * j:32 * j + 16]] * 4
                        + [iv[:, 32 * j + 16:32 * j + 32]] * 4, axis=1)
        for j in range(4)
    ]
    inv = jnp.stack(pieces, axis=1).reshape(m, 128)
    return jnp.maximum((agg * inv + r_ref[...]) * a_ref[...] + cv_ref[...],
                       0.0)


def _node_specs(extra):
    return [
        pl.BlockSpec((_NC, _BR, 128), lambda i: (0, i, 0)),
        pl.BlockSpec((_NC, _BR // 4, 128), lambda i: (0, i, 0)),
        pl.BlockSpec((_BR, 128), lambda i: (i, 0)),
        pl.BlockSpec((1, 128), lambda i: (0, 0)),
        pl.BlockSpec((1, 128), lambda i: (0, 0)),
    ] + extra


def _mid_layer(p, cnt, r, a, cv, wl):
    m = r.shape[0]

    def body(p_ref, c_ref, r_ref, a_ref, cv_ref, wl_ref, y_ref, h_ref):
        hid = _combine(p_ref, c_ref, r_ref, a_ref, cv_ref, _BR, _BR // 4)
        y_ref[...] = lax.dot(hid, wl_ref[...], precision=_HIGH)
        h_ref[...] = hid

    return pl.pallas_call(
        body,
        grid=_grid(m),
        in_specs=_node_specs([pl.BlockSpec(wl.shape, lambda i: (0, 0))]),
        out_specs=(pl.BlockSpec((_BR, 128), lambda i: (i, 0)),
                   pl.BlockSpec((_BR, 128), lambda i: (i, 0))),
        out_shape=(jax.ShapeDtypeStruct((m, 128), jnp.float32),
                   jax.ShapeDtypeStruct((m, 128), jnp.float32)),
    )(p, cnt, r, a, cv, wl)


def _final_layer(p, cnt, r, a, cv, wo, bo):
    m = r.shape[0]

    def body(p_ref, c_ref, r_ref, a_ref, cv_ref, wo_ref, bo_ref, o_ref):
        hid = _combine(p_ref, c_ref, r_ref, a_ref, cv_ref, _BR, _BR // 4)
        t = hid * wo_ref[...]
        lo = jnp.sum(t[:, :64], axis=1, keepdims=True)
        hi = jnp.sum(t[:, 64:], axis=1, keepdims=True)
        o_ref[...] = jnp.concatenate([lo, hi], axis=1) + bo_ref[...]

    return pl.pallas_call(
        body,
        grid=_grid(m),
        in_specs=_node_specs([pl.BlockSpec((1, 128), lambda i: (0, 0)),
                              pl.BlockSpec((1, 1), lambda i: (0, 0))]),
        out_specs=pl.BlockSpec((_BR, 2), lambda i: (i, 0)),
        out_shape=jax.ShapeDtypeStruct((m, 2), jnp.float32),
    )(p, cnt, r, a, cv, wo, bo)


def kernel(x, edge_index, Wl1, bl1, Wr1, Wl2, bl2, Wr2,
           g1, b1, rm1, rv1, g2, b2, rm2, rv2, Wo, bo):
    n, d = x.shape
    e = edge_index.shape[1]
    h = Wl1.shape[0]

    e_pad = _round_up(e, _NW * _CHUNK * _NBUF)
    n_pad = _round_up(n, _NS * _ZB)
    if e_pad > e and n_pad == n:
        n_pad += _NS * _ZB
    src = edge_index[0]
    dst = edge_index[1]
    if e_pad > e:
        pad = e_pad - e
        ar = jnp.arange(pad, dtype=jnp.int32)
        src = jnp.concatenate([src, ar % n])
        dst = jnp.concatenate([dst, n + ar % (n_pad - n)])
    nchunk = e_pad // (_NW * _CHUNK)
    src = src.reshape(_NW, nchunk, _CHUNK)
    dst = dst.reshape(_NW, nchunk, _CHUNK)

    a1 = g1 / jnp.sqrt(rv1 + 1e-5)
    c1 = b1 + (bl1 - rm1) * a1
    a2 = g2 / jnp.sqrt(rv2 + 1e-5)
    c2 = b2 + (bl2 - rm2) * a2

    sc1 = _make_sc_segsum(n, h, n_pad, e_pad, with_cnt=True)
    sc2 = _make_sc_segsum(n, h, n_pad, e_pad, with_cnt=False)

    bd = block_diag
    two = lambda v: jnp.concatenate([v, v]).reshape(1, 2 * h)
    x_pair = x.reshape(n // 2, 2 * d)

    y1p = _proj1(x_pair, bd(Wl1.T, Wl1.T))
    p1, cnt = sc1(y1p.reshape(n, h), src, dst)
    r1p = _proj1(x_pair, bd(Wr1.T, Wr1.T))
    p1p = p1.reshape(_NC, n_pad // 2, 128)
    cntp = cnt.reshape(_NC, n_pad * _L // 128, 128)
    y2p, hmid = _mid_layer(p1p, cntp, r1p, two(a1), two(c1),
                           bd(Wl2.T, Wl2.T))
    (p2,) = sc2(y2p.reshape(n, h), src, dst)
    r2p = _proj1(hmid, bd(Wr2.T, Wr2.T))
    p2p = p2.reshape(_NC, n_pad // 2, 128)
    out = _final_layer(p2p, cntp, r2p, two(a2), two(c2),
                       two(Wo.reshape(h)), bo.reshape(1, 1))
    return out.reshape(n, 1)
